# baseline (speedup 1.0000x reference)
import functools

import jax
import jax.numpy as jnp
from jax import lax
from jax.experimental import pallas as pl
from jax.experimental.pallas import tpu as pltpu
from jax.experimental.pallas import tpu_sc as plsc

NUM_FIELDS = 26
EMBED_DIM = 16
BATCH = 16384
FIELD_SIZE = 100000

NUM_CORES = 2
NUM_SUBCORES = 16
NUM_WORKERS = NUM_CORES * NUM_SUBCORES
B_PER_W = BATCH // NUM_WORKERS
CHUNK_B = 64
NUM_CHUNKS = B_PER_W // CHUNK_B
CHUNK_IDX = CHUNK_B * NUM_FIELDS
IDX_VECS = CHUNK_IDX // 16
GATHER_N = 128
NUM_GATHERS = CHUNK_IDX // GATHER_N


def _fm_body(x_hbm, v_hbm, w_hbm, b_hbm, out_hbm,
             xbuf, idxv, pat, rows, wv, wide, outw, bv, outb, semv, semw):
    wid = lax.axis_index("s") * NUM_CORES + lax.axis_index("c")
    iota = jax.lax.iota(jnp.int32, 16)

    pltpu.sync_copy(b_hbm, bv)

    def pat_body(j, _):
        pos = j * 16 + iota
        pat[pl.ds(j * 16, 16)] = lax.rem(pos, NUM_FIELDS) * FIELD_SIZE
        return 0
    lax.fori_loop(0, IDX_VECS, pat_body, 0)

    flat_base = wid * (B_PER_W * NUM_FIELDS)
    out_base = wid * B_PER_W

    def chunk_body(c, _):
        pltpu.sync_copy(x_hbm.at[pl.ds(flat_base + c * CHUNK_IDX, CHUNK_IDX)],
                        xbuf)

        def add_body(j, _):
            sl = pl.ds(j * 16, 16)
            idxv[sl] = xbuf[sl] + pat[sl]
            return 0
        lax.fori_loop(0, IDX_VECS, add_body, 0)

        handles = []
        for g in range(NUM_GATHERS):
            isl = pl.ds(g * GATHER_N, GATHER_N)
            handles.append(pltpu.async_copy(
                v_hbm.at[idxv.at[isl]], rows.at[isl], semv))
            handles.append(pltpu.async_copy(
                w_hbm.at[idxv.at[isl]], wv.at[isl], semw))
        for h in handles:
            h.wait()

        lin_mask = iota < (NUM_FIELDS - 16)

        def elem_body(e, _):
            base = e * NUM_FIELDS
            acc = jnp.zeros((16,), jnp.float32)
            acc2 = jnp.zeros((16,), jnp.float32)
            for f in range(NUM_FIELDS):
                v = rows[base + f, :]
                acc = acc + v
                acc2 = acc2 + v * v
            w0 = wv[pl.ds(base, 16)]
            w1 = jnp.where(lin_mask, wv[pl.ds(base + 16, 16)], 0.0)
            u = 0.5 * (acc * acc - acc2) + w0 + w1
            for sh in (8, 4, 2, 1):
                wide[pl.ds(0, 16)] = u
                u = u + wide[pl.ds(sh, 16)]
            outw[pl.ds(e, 16)] = u
            return 0
        lax.fori_loop(0, CHUNK_B, elem_body, 0)

        for g16 in range(CHUNK_B // 16):
            z = outw[pl.ds(g16 * 16, 16)] + bv[...]
            outb[pl.ds(g16 * 16, 16)] = 1.0 / (1.0 + jnp.exp(-z))

        pltpu.sync_copy(outb, out_hbm.at[pl.ds(out_base + c * CHUNK_B, CHUNK_B)])
        return 0

    lax.fori_loop(0, NUM_CHUNKS, chunk_body, 0)


_fm_call = functools.partial(
    pl.kernel,
    mesh=plsc.VectorSubcoreMesh(core_axis_name="c", subcore_axis_name="s"),
    out_type=jax.ShapeDtypeStruct((BATCH,), jnp.float32),
    compiler_params=pltpu.CompilerParams(use_tc_tiling_on_sc=False),
    scratch_types=[
        pltpu.VMEM((CHUNK_IDX,), jnp.int32),
        pltpu.VMEM((CHUNK_IDX,), jnp.int32),
        pltpu.VMEM((CHUNK_IDX,), jnp.int32),
        pltpu.VMEM((CHUNK_IDX, EMBED_DIM), jnp.float32),
        pltpu.VMEM((CHUNK_IDX + 16,), jnp.float32),
        pltpu.VMEM((32,), jnp.float32),
        pltpu.VMEM((CHUNK_B + 16,), jnp.float32),
        pltpu.VMEM((16,), jnp.float32),
        pltpu.VMEM((CHUNK_B,), jnp.float32),
        pltpu.SemaphoreType.DMA,
        pltpu.SemaphoreType.DMA,
    ],
)(_fm_body)


@jax.jit
def kernel(x, V, w, b):
    x32 = x.astype(jnp.int32).reshape(-1)
    w_flat = w.reshape(-1)
    b16 = jnp.broadcast_to(b, (16,))
    return _fm_call(x32, V, w_flat, b16)


# speedup vs baseline: 1.0176x; 1.0176x over previous
import functools

import jax
import jax.numpy as jnp
from jax import lax
from jax.experimental import pallas as pl
from jax.experimental.pallas import tpu as pltpu
from jax.experimental.pallas import tpu_sc as plsc

NUM_FIELDS = 26
EMBED_DIM = 16
BATCH = 16384
FIELD_SIZE = 100000

NUM_CORES = 2
NUM_SUBCORES = 16
NUM_WORKERS = NUM_CORES * NUM_SUBCORES
B_PER_W = BATCH // NUM_WORKERS
CHUNK_B = 64
NUM_CHUNKS = B_PER_W // CHUNK_B
CHUNK_IDX = CHUNK_B * NUM_FIELDS
IDX_VECS = CHUNK_IDX // 16
GATHER_N = 128
NUM_GATHERS = CHUNK_IDX // GATHER_N


def _fm_body(x_hbm, v_hbm, w_hbm, b_hbm, out_hbm,
             xbuf, idxv0, idxv1, pat, rows0, rows1, wv0, wv1,
             wide, outw, bv, outb, semv, semw):
    wid = lax.axis_index("s") * NUM_CORES + lax.axis_index("c")
    iota = jax.lax.iota(jnp.int32, 16)

    pltpu.sync_copy(b_hbm, bv)

    def pat_body(j, _):
        pos = j * 16 + iota
        pat[pl.ds(j * 16, 16)] = lax.rem(pos, NUM_FIELDS) * FIELD_SIZE
        return 0
    lax.fori_loop(0, IDX_VECS, pat_body, 0)

    flat_base = wid * (B_PER_W * NUM_FIELDS)
    out_base = wid * B_PER_W

    idxvs = (idxv0, idxv1)
    rowss = (rows0, rows1)
    wvs = (wv0, wv1)
    lin_mask = iota < (NUM_FIELDS - 16)

    def stage(c, buf):
        idxv, rows, wv = idxvs[buf], rowss[buf], wvs[buf]
        pltpu.sync_copy(x_hbm.at[pl.ds(flat_base + c * CHUNK_IDX, CHUNK_IDX)],
                        xbuf)

        def add_body(j, _):
            sl = pl.ds(j * 16, 16)
            idxv[sl] = xbuf[sl] + pat[sl]
            return 0
        lax.fori_loop(0, IDX_VECS, add_body, 0)

        handles = []
        for g in range(NUM_GATHERS):
            isl = pl.ds(g * GATHER_N, GATHER_N)
            handles.append(pltpu.async_copy(
                v_hbm.at[idxv.at[isl]], rows.at[isl], semv))
            handles.append(pltpu.async_copy(
                w_hbm.at[idxv.at[isl]], wv.at[isl], semw))
        return handles

    def compute(c, buf):
        rows, wv = rowss[buf], wvs[buf]

        def elem_body(e, _):
            base = e * NUM_FIELDS
            acc = jnp.zeros((16,), jnp.float32)
            acc2 = jnp.zeros((16,), jnp.float32)
            for f in range(NUM_FIELDS):
                v = rows[base + f, :]
                acc = acc + v
                acc2 = acc2 + v * v
            w0 = wv[pl.ds(base, 16)]
            w1 = jnp.where(lin_mask, wv[pl.ds(base + 16, 16)], 0.0)
            u = 0.5 * (acc * acc - acc2) + w0 + w1
            for sh in (8, 4, 2, 1):
                wide[pl.ds(0, 16)] = u
                u = u + wide[pl.ds(sh, 16)]
            outw[pl.ds(e, 16)] = u
            return 0
        lax.fori_loop(0, CHUNK_B, elem_body, 0)

        for g16 in range(CHUNK_B // 16):
            z = outw[pl.ds(g16 * 16, 16)] + bv[...]
            outb[pl.ds(g16 * 16, 16)] = 1.0 / (1.0 + jnp.exp(-z))

        pltpu.sync_copy(outb, out_hbm.at[pl.ds(out_base + c * CHUNK_B, CHUNK_B)])

    pending = stage(0, 0)
    for c in range(NUM_CHUNKS):
        buf = c % 2
        cur = pending
        if c + 1 < NUM_CHUNKS:
            pending = stage(c + 1, 1 - buf)
        for h in cur:
            h.wait()
        compute(c, buf)


_fm_call = functools.partial(
    pl.kernel,
    mesh=plsc.VectorSubcoreMesh(core_axis_name="c", subcore_axis_name="s"),
    out_type=jax.ShapeDtypeStruct((BATCH,), jnp.float32),
    compiler_params=pltpu.CompilerParams(use_tc_tiling_on_sc=False),
    scratch_types=[
        pltpu.VMEM((CHUNK_IDX,), jnp.int32),
        pltpu.VMEM((CHUNK_IDX,), jnp.int32),
        pltpu.VMEM((CHUNK_IDX,), jnp.int32),
        pltpu.VMEM((CHUNK_IDX,), jnp.int32),
        pltpu.VMEM((CHUNK_IDX, EMBED_DIM), jnp.float32),
        pltpu.VMEM((CHUNK_IDX, EMBED_DIM), jnp.float32),
        pltpu.VMEM((CHUNK_IDX + 16,), jnp.float32),
        pltpu.VMEM((CHUNK_IDX + 16,), jnp.float32),
        pltpu.VMEM((32,), jnp.float32),
        pltpu.VMEM((CHUNK_B + 16,), jnp.float32),
        pltpu.VMEM((16,), jnp.float32),
        pltpu.VMEM((CHUNK_B,), jnp.float32),
        pltpu.SemaphoreType.DMA,
        pltpu.SemaphoreType.DMA,
    ],
)(_fm_body)


@jax.jit
def kernel(x, V, w, b):
    x32 = x.astype(jnp.int32).reshape(-1)
    b16 = jnp.broadcast_to(b, (16,))
    return _fm_call(x32, V, w.reshape(-1), b16)
